# Initial kernel scaffold; baseline (speedup 1.0000x reference)
import jax
import jax.numpy as jnp
from jax.experimental import pallas as pl


def kernel(image, pixel_W, pixel_b, edge_W, edge_b, edges):
    raise NotImplementedError("write your pallas kernel here")


# speedup vs baseline: 1.6445x; 1.6445x over previous
import functools

import jax
import jax.numpy as jnp
from jax import lax
from jax.experimental import pallas as pl
from jax.experimental.pallas import tpu as pltpu
from jax.experimental.pallas import tpu_sc as plsc

N = 224 * 224
F = 128
C = 21
CP = 32
E = 2 * 224 * 224 - 2 * 224
CHUNK = 128
NW = 32
EPW = 3200
EP = NW * EPW
IPW = EPW // CHUNK
ROWS_BLK = 1792


def _mm_body(x_ref, wp_ref, bp_ref, wa_ref, ba_ref, wb_ref,
             p_ref, a_ref, b_ref):
    x = x_ref[...]
    p_ref[...] = jnp.dot(x, wp_ref[...],
                         preferred_element_type=jnp.float32) + bp_ref[...]
    a_ref[...] = jnp.dot(x, wa_ref[...],
                         preferred_element_type=jnp.float32) + ba_ref[...]
    b_ref[...] = jnp.dot(x, wb_ref[...], preferred_element_type=jnp.float32)


def _tc_matmuls(x, wp, bp, wa, ba, wb):
    grid = (N // ROWS_BLK,)
    return pl.pallas_call(
        _mm_body,
        grid=grid,
        in_specs=[
            pl.BlockSpec((ROWS_BLK, F), lambda i: (i, 0)),
            pl.BlockSpec((F, C), lambda i: (0, 0)),
            pl.BlockSpec((1, C), lambda i: (0, 0)),
            pl.BlockSpec((F, CP), lambda i: (0, 0)),
            pl.BlockSpec((1, CP), lambda i: (0, 0)),
            pl.BlockSpec((F, CP), lambda i: (0, 0)),
        ],
        out_specs=[
            pl.BlockSpec((ROWS_BLK, C), lambda i: (i, 0)),
            pl.BlockSpec((ROWS_BLK, CP), lambda i: (i, 0)),
            pl.BlockSpec((ROWS_BLK, CP), lambda i: (i, 0)),
        ],
        out_shape=[
            jax.ShapeDtypeStruct((N, C), jnp.float32),
            jax.ShapeDtypeStruct((N, CP), jnp.float32),
            jax.ShapeDtypeStruct((N, CP), jnp.float32),
        ],
    )(x, wp, bp, wa, ba, wb)


def _sc_body(a_hbm, b_hbm, src_hbm, dst_hbm, out_hbm,
             src_v, dst_v, buf, sem_a, sem_b):
    wid = lax.axis_index("s") * 2 + lax.axis_index("c")
    base_e = wid * EPW
    pltpu.sync_copy(src_hbm.at[pl.ds(base_e, EPW)], src_v)
    pltpu.sync_copy(dst_hbm.at[pl.ds(base_e, EPW)], dst_v)

    def chunk(ci, carry):
        idx_s = src_v.at[pl.ds(ci * CHUNK, CHUNK)]
        idx_d = dst_v.at[pl.ds(ci * CHUNK, CHUNK)]
        pltpu.async_copy(a_hbm.at[idx_s], buf, sem_a).wait()
        pltpu.async_copy(b_hbm.at[idx_d], buf, sem_b, add=True).wait()
        pltpu.sync_copy(buf, out_hbm.at[pl.ds(base_e + ci * CHUNK, CHUNK)])
        return carry

    lax.fori_loop(0, IPW, chunk, 0)


def _sc_gather_add(a_tab, b_tab, src2d, dst2d):
    mesh = plsc.VectorSubcoreMesh(core_axis_name="c", subcore_axis_name="s")
    fn = functools.partial(
        pl.kernel,
        out_type=jax.ShapeDtypeStruct((EP, CP), jnp.float32),
        mesh=mesh,
        compiler_params=pltpu.CompilerParams(use_tc_tiling_on_sc=False),
        scratch_types=[
            pltpu.VMEM((EPW,), jnp.int32),
            pltpu.VMEM((EPW,), jnp.int32),
            pltpu.VMEM((CHUNK, CP), jnp.float32),
            pltpu.SemaphoreType.DMA,
            pltpu.SemaphoreType.DMA,
        ],
    )(_sc_body)
    return fn(a_tab, b_tab, src2d, dst2d)


def kernel(image, pixel_W, pixel_b, edge_W, edge_b, edges):
    x = image.reshape(N, F)
    wsrc = edge_W[:F]
    wdst = edge_W[F:]
    pad = jnp.zeros((F, CP - C), jnp.float32)
    wa = jnp.concatenate([wsrc, pad], axis=1)
    wb = jnp.concatenate([wdst, pad], axis=1)
    bp = pixel_b.reshape(1, C)
    ba = jnp.concatenate([edge_b, jnp.zeros((CP - C,), jnp.float32)])
    ba = ba.reshape(1, CP)

    p, a_tab, b_tab = _tc_matmuls(x, pixel_W, bp, wa, ba, wb)

    src = edges[:, 0].astype(jnp.int32)
    dst = edges[:, 1].astype(jnp.int32)
    zpad = jnp.zeros((EP - E,), jnp.int32)
    src_flat = jnp.concatenate([src, zpad])
    dst_flat = jnp.concatenate([dst, zpad])

    e_pad = _sc_gather_add(a_tab, b_tab, src_flat, dst_flat)
    return p, e_pad[:E, :C]


# speedup vs baseline: 1.7657x; 1.0737x over previous
import functools

import jax
import jax.numpy as jnp
from jax import lax
from jax.experimental import pallas as pl
from jax.experimental.pallas import tpu as pltpu
from jax.experimental.pallas import tpu_sc as plsc

N = 224 * 224
F = 128
C = 21
CP = 32
E = 2 * 224 * 224 - 2 * 224
CHUNK = 128
NW = 32
EPW = 3200
EP = NW * EPW
IPW = EPW // CHUNK
ROWS_BLK = 1792


def _mm_body(x_ref, wp_ref, bp_ref, wa_ref, ba_ref, wb_ref,
             p_ref, a_ref, b_ref):
    x = x_ref[...]
    p_ref[...] = jnp.dot(x, wp_ref[...],
                         preferred_element_type=jnp.float32) + bp_ref[...]
    a_ref[...] = jnp.dot(x, wa_ref[...],
                         preferred_element_type=jnp.float32) + ba_ref[...]
    b_ref[...] = jnp.dot(x, wb_ref[...], preferred_element_type=jnp.float32)


def _tc_matmuls(x, wp, bp, wa, ba, wb):
    grid = (N // ROWS_BLK,)
    return pl.pallas_call(
        _mm_body,
        grid=grid,
        in_specs=[
            pl.BlockSpec((ROWS_BLK, F), lambda i: (i, 0)),
            pl.BlockSpec((F, C), lambda i: (0, 0)),
            pl.BlockSpec((1, C), lambda i: (0, 0)),
            pl.BlockSpec((F, CP), lambda i: (0, 0)),
            pl.BlockSpec((1, CP), lambda i: (0, 0)),
            pl.BlockSpec((F, CP), lambda i: (0, 0)),
        ],
        out_specs=[
            pl.BlockSpec((ROWS_BLK, C), lambda i: (i, 0)),
            pl.BlockSpec((ROWS_BLK, CP), lambda i: (i, 0)),
            pl.BlockSpec((ROWS_BLK, CP), lambda i: (i, 0)),
        ],
        out_shape=[
            jax.ShapeDtypeStruct((N, C), jnp.float32),
            jax.ShapeDtypeStruct((N, CP), jnp.float32),
            jax.ShapeDtypeStruct((N, CP), jnp.float32),
        ],
    )(x, wp, bp, wa, ba, wb)


def _sc_body(a_hbm, b_hbm, src_hbm, dst_hbm, out_hbm,
             src_v, dst_v, buf, sem_a, sem_b):
    wid = lax.axis_index("s") * 2 + lax.axis_index("c")
    base_e = wid * EPW
    pltpu.sync_copy(src_hbm.at[pl.ds(base_e, EPW)], src_v)
    pltpu.sync_copy(dst_hbm.at[pl.ds(base_e, EPW)], dst_v)

    def make_fire(tab, idx_v, sem, add):
        def fire(ci, carry):
            off = pl.multiple_of(ci * CHUNK, CHUNK)
            idx = idx_v.at[pl.ds(off, CHUNK)]
            pltpu.async_copy(tab.at[idx], buf.at[pl.ds(off, CHUNK)],
                             sem, add=add)
            return carry
        return fire

    lax.fori_loop(0, IPW, make_fire(a_hbm, src_v, sem_a, False), 0)
    pltpu.make_async_copy(a_hbm.at[src_v], buf, sem_a).wait()
    lax.fori_loop(0, IPW, make_fire(b_hbm, dst_v, sem_b, True), 0)
    pltpu.make_async_copy(b_hbm.at[dst_v], buf, sem_b).wait()
    pltpu.sync_copy(buf, out_hbm.at[pl.ds(base_e, EPW)])


def _sc_gather_add(a_tab, b_tab, src2d, dst2d):
    mesh = plsc.VectorSubcoreMesh(core_axis_name="c", subcore_axis_name="s")
    fn = functools.partial(
        pl.kernel,
        out_type=jax.ShapeDtypeStruct((EP, CP), jnp.float32),
        mesh=mesh,
        compiler_params=pltpu.CompilerParams(use_tc_tiling_on_sc=False),
        scratch_types=[
            pltpu.VMEM((EPW,), jnp.int32),
            pltpu.VMEM((EPW,), jnp.int32),
            pltpu.VMEM((EPW, CP), jnp.float32),
            pltpu.SemaphoreType.DMA,
            pltpu.SemaphoreType.DMA,
        ],
    )(_sc_body)
    return fn(a_tab, b_tab, src2d, dst2d)


def kernel(image, pixel_W, pixel_b, edge_W, edge_b, edges):
    x = image.reshape(N, F)
    wsrc = edge_W[:F]
    wdst = edge_W[F:]
    pad = jnp.zeros((F, CP - C), jnp.float32)
    wa = jnp.concatenate([wsrc, pad], axis=1)
    wb = jnp.concatenate([wdst, pad], axis=1)
    bp = pixel_b.reshape(1, C)
    ba = jnp.concatenate([edge_b, jnp.zeros((CP - C,), jnp.float32)])
    ba = ba.reshape(1, CP)

    p, a_tab, b_tab = _tc_matmuls(x, pixel_W, bp, wa, ba, wb)

    src = edges[:, 0].astype(jnp.int32)
    dst = edges[:, 1].astype(jnp.int32)
    zpad = jnp.zeros((EP - E,), jnp.int32)
    src_flat = jnp.concatenate([src, zpad])
    dst_flat = jnp.concatenate([dst, zpad])

    e_pad = _sc_gather_add(a_tab, b_tab, src_flat, dst_flat)
    return p, e_pad[:E, :C]
